# baseline (speedup 1.0000x reference)
import functools

import jax
import jax.numpy as jnp
from jax import lax
from jax.experimental import pallas as pl
from jax.experimental.pallas import tpu as pltpu
from jax.experimental.pallas import tpu_sc as plsc

VOCAB = 1000000
DIM = 64
N = 16384

NC = 2
NS = 16
NW = NC * NS
B_W = N // NW
CHUNK = 128
NCHUNK = B_W // CHUNK

W_BLK = 32768
NB = -(-VOCAB // W_BLK)
UPAD = NB * W_BLK


def _usum_body(t_ref, w_ref, u_ref):
    for g in range(W_BLK // 128):
        sl = pl.ds(g * 128, 128)
        u_ref[0, sl] = jnp.sum(t_ref[:, sl] * w_ref[...], axis=0)


_usum = pl.pallas_call(
    _usum_body,
    grid=(NB,),
    in_specs=[
        pl.BlockSpec((DIM, W_BLK), lambda b: (0, b)),
        pl.BlockSpec((DIM, 1), lambda b: (0, 0)),
    ],
    out_specs=pl.BlockSpec((1, W_BLK), lambda b: (0, b)),
    out_shape=jax.ShapeDtypeStruct((1, UPAD), jnp.float32),
)


@functools.partial(
    pl.kernel,
    mesh=plsc.VectorSubcoreMesh(core_axis_name="c", subcore_axis_name="s"),
    out_type=jax.ShapeDtypeStruct((NW, 16), jnp.float32),
    scratch_types=[
        pltpu.VMEM((NCHUNK, CHUNK), jnp.int32),
        pltpu.VMEM((B_W,), jnp.float32),
        pltpu.VMEM((16,), jnp.float32),
        pltpu.SemaphoreType.DMA,
    ],
)
def _gather_sum(idx_hbm, u_hbm, out_hbm, idx_v, vals_v, acc_v, sem):
    wid = lax.axis_index("s") * NC + lax.axis_index("c")
    pltpu.sync_copy(idx_hbm.at[wid], idx_v)
    copies = [
        pltpu.async_copy(
            u_hbm.at[idx_v.at[j]],
            vals_v.at[pl.ds(j * CHUNK, CHUNK)],
            sem,
        )
        for j in range(NCHUNK)
    ]
    for c in copies:
        c.wait()

    def body(i, acc):
        return acc + vals_v[pl.ds(i * 16, 16)]

    acc = lax.fori_loop(0, B_W // 16, body, jnp.zeros((16,), jnp.float32))
    acc_v[...] = acc
    pltpu.sync_copy(acc_v, out_hbm.at[wid])


def _finalize_body(p_ref, o_ref):
    o_ref[...] = jnp.sum(p_ref[...]).reshape(1, 1) * (1.0 / N)


_finalize = pl.pallas_call(
    _finalize_body,
    out_shape=jax.ShapeDtypeStruct((1, 1), jnp.float32),
)


def kernel(word_idxs, table, weights):
    idx = word_idxs.astype(jnp.int32).reshape(NW, NCHUNK, CHUNK)
    u = _usum(table.T, weights.reshape(DIM, 1)).reshape(UPAD)
    partials = _gather_sum(idx, u)
    prod = _finalize(partials)
    return jnp.reshape(prod, ())


# speedup vs baseline: 1.1589x; 1.1589x over previous
import functools

import jax
import jax.numpy as jnp
from jax import lax
from jax.experimental import pallas as pl
from jax.experimental.pallas import tpu as pltpu
from jax.experimental.pallas import tpu_sc as plsc

VOCAB = 1000000
DIM = 64
N = 16384

NC = 2
NS = 16
NW = NC * NS
B_W = N // NW
CHUNK = 128
NCHUNK = B_W // CHUNK

W_BLK = 32768
NB = -(-VOCAB // W_BLK)
UPAD = NB * W_BLK


def _usum_body(t_ref, w_ref, u_ref, a8_ref):
    for g in range(W_BLK // 128):
        sl = pl.ds(g * 128, 128)
        s8 = None
        for r in range(DIM // 8):
            p = t_ref[pl.ds(8 * r, 8), sl] * w_ref[pl.ds(8 * r, 8), :]
            s8 = p if s8 is None else s8 + p
        a8_ref[:, sl] = s8
    for g in range(W_BLK // 128):
        sl = pl.ds(g * 128, 128)
        a8_ref[pl.ds(0, 4), sl] = (
            a8_ref[pl.ds(0, 4), sl] + a8_ref[pl.ds(4, 4), sl])
        a8_ref[pl.ds(0, 2), sl] = (
            a8_ref[pl.ds(0, 2), sl] + a8_ref[pl.ds(2, 2), sl])
        u_ref[0, sl] = a8_ref[0, sl] + a8_ref[1, sl]


_usum = pl.pallas_call(
    _usum_body,
    grid=(NB,),
    in_specs=[
        pl.BlockSpec((DIM, W_BLK), lambda b: (0, b)),
        pl.BlockSpec((DIM, 128), lambda b: (0, 0)),
    ],
    out_specs=pl.BlockSpec((1, W_BLK), lambda b: (0, b)),
    out_shape=jax.ShapeDtypeStruct((1, UPAD), jnp.float32),
    scratch_shapes=[pltpu.VMEM((8, W_BLK), jnp.float32)],
)


@functools.partial(
    pl.kernel,
    mesh=plsc.VectorSubcoreMesh(core_axis_name="c", subcore_axis_name="s"),
    out_type=jax.ShapeDtypeStruct((NW, 16), jnp.float32),
    scratch_types=[
        pltpu.VMEM((NCHUNK, CHUNK), jnp.int32),
        pltpu.VMEM((B_W,), jnp.float32),
        pltpu.VMEM((16,), jnp.float32),
        pltpu.SemaphoreType.DMA,
    ],
)
def _gather_sum(idx_hbm, u_hbm, out_hbm, idx_v, vals_v, acc_v, sem):
    wid = lax.axis_index("s") * NC + lax.axis_index("c")
    pltpu.sync_copy(idx_hbm.at[wid], idx_v)
    copies = [
        pltpu.async_copy(
            u_hbm.at[idx_v.at[j]],
            vals_v.at[pl.ds(j * CHUNK, CHUNK)],
            sem,
        )
        for j in range(NCHUNK)
    ]
    for c in copies:
        c.wait()

    def body(i, acc):
        return acc + vals_v[pl.ds(i * 16, 16)]

    acc = lax.fori_loop(0, B_W // 16, body, jnp.zeros((16,), jnp.float32))
    acc_v[...] = acc
    pltpu.sync_copy(acc_v, out_hbm.at[wid])


def _finalize_body(p_ref, o_ref):
    o_ref[...] = jnp.sum(p_ref[...]).reshape(1, 1) * (1.0 / N)


_finalize = pl.pallas_call(
    _finalize_body,
    out_shape=jax.ShapeDtypeStruct((1, 1), jnp.float32),
)


def kernel(word_idxs, table, weights):
    idx = word_idxs.astype(jnp.int32).reshape(NW, NCHUNK, CHUNK)
    wb = jnp.broadcast_to(weights.reshape(DIM, 1), (DIM, 128))
    u = _usum(table.T, wb).reshape(UPAD)
    partials = _gather_sum(idx, u)
    prod = _finalize(partials)
    return jnp.reshape(prod, ())
